# Initial kernel scaffold; baseline (speedup 1.0000x reference)
import jax
import jax.numpy as jnp
from jax.experimental import pallas as pl


def kernel(user_idx, item_idx, user_emb, item_emb):
    raise NotImplementedError("write your pallas kernel here")


# speedup vs baseline: 3.4136x; 3.4136x over previous
import functools

import jax
import jax.numpy as jnp
import numpy as np
from jax import lax
from jax.experimental import pallas as pl
from jax.experimental.pallas import tpu as pltpu
from jax.experimental.pallas import tpu_sc as plsc

B = 16384
K = 200
D = 32
TAU = 0.5
NW = 32
UPW = B // NW
KB = 13
L = 16


_TAKE_DNUMS = lax.GatherDimensionNumbers(
    offset_dims=(), collapsed_slice_dims=(0,), start_index_map=(0,))


def _lane_take(x, idx):
    return lax.gather(x, jnp.asarray(idx).reshape(16, 1), _TAKE_DNUMS, (1,),
                      mode=lax.GatherScatterMode.PROMISE_IN_BOUNDS)


def _tree_sum(x, iota):
    for s in (1, 2, 4, 8):
        x = x + _lane_take(x, jnp.bitwise_xor(iota, s))
    return x


def _rsqrt_nr(x):
    i = lax.bitcast_convert_type(x, jnp.int32)
    i = jnp.int32(0x5F3759DF) - lax.shift_right_arithmetic(i, 1)
    y = lax.bitcast_convert_type(i, jnp.float32)
    half = x * 0.5
    for _ in range(3):
        y = y * (1.5 - half * y * y)
    return y


def _sc_body(uidx_hbm, iidx_hbm, uemb_hbm, iemb_hbm, out_hbm,
             uidx_s, urows_s, iidx_s, rows_s, score_s, sem):
    wid = lax.axis_index("s") * 2 + lax.axis_index("c")
    b0 = wid * UPW

    pltpu.sync_copy(uidx_hbm.at[wid], uidx_s)
    for j in range(4):
        pltpu.async_copy(uemb_hbm.at[uidx_s.at[j]],
                         urows_s.at[pl.ds(j * 128, 128)], sem).wait()

    iota = lax.iota(jnp.int32, L)

    def user_body(t, carry):
        b = b0 + t
        pltpu.sync_copy(iidx_hbm.at[b], iidx_s)
        for j in range(2):
            pltpu.async_copy(iemb_hbm.at[iidx_s.at[j]],
                             rows_s.at[pl.ds(j * 100, 100)], sem).wait()

        u0 = urows_s[t, pl.ds(0, L)]
        u1 = urows_s[t, pl.ds(L, L)]
        n2u = _tree_sum(u0 * u0 + u1 * u1, iota)
        cu = _rsqrt_nr(jnp.maximum(n2u, 1e-24)) * (1.0 / TAU)

        for blk in range(KB):
            rows = jnp.minimum(iota + blk * L, K - 1)
            acc = jnp.zeros((L,), jnp.float32)
            n2 = jnp.zeros((L,), jnp.float32)
            for d in range(D):
                x = plsc.load_gather(rows_s, [rows, jnp.full((L,), d, jnp.int32)])
                bcast = jnp.full((L,), d % L, jnp.int32)
                ud = _lane_take(u0 if d < L else u1, bcast)
                acc = acc + ud * x
                n2 = n2 + x * x
            r = _rsqrt_nr(jnp.maximum(n2, 1e-24))
            score_s[pl.ds(blk * L, L)] = acc * r * cu
        pltpu.sync_copy(score_s.at[pl.ds(0, K)], out_hbm.at[b])
        return carry

    lax.fori_loop(0, UPW, user_body, 0)


@jax.jit
def _launch(user_idx, item_idx, uemb, iemb):
    uidx = user_idx.reshape(NW, 4, 128).astype(jnp.int32)
    iidx = item_idx.reshape(B, 2, 100).astype(jnp.int32)
    mesh = plsc.VectorSubcoreMesh(core_axis_name="c", subcore_axis_name="s")
    kern = functools.partial(
        pl.kernel,
        out_type=jax.ShapeDtypeStruct((B, K), jnp.float32),
        mesh=mesh,
        compiler_params=pltpu.CompilerParams(
            needs_layout_passes=False, use_tc_tiling_on_sc=False),
        scratch_types=[
            pltpu.VMEM((4, 128), jnp.int32),
            pltpu.VMEM((UPW, D), jnp.float32),
            pltpu.VMEM((2, 100), jnp.int32),
            pltpu.VMEM((K, D), jnp.float32),
            pltpu.VMEM((KB * L,), jnp.float32),
            pltpu.SemaphoreType.DMA,
        ],
    )(_sc_body)
    return kern(uidx, iidx, uemb, iemb)


def kernel(user_idx, item_idx, user_emb, item_emb):
    return _launch(user_idx, item_idx, user_emb, item_emb)


# speedup vs baseline: 4.4270x; 1.2969x over previous
import functools

import jax
import jax.numpy as jnp
from jax import lax
from jax.experimental import pallas as pl
from jax.experimental.pallas import tpu as pltpu
from jax.experimental.pallas import tpu_sc as plsc

B = 16384
K = 200
D = 32
TAU = 0.5
NW = 32
UPW = B // NW
KB = 13
L = 16

_TAKE_DNUMS = lax.GatherDimensionNumbers(
    offset_dims=(), collapsed_slice_dims=(0,), start_index_map=(0,))


def _lane_take(x, idx):
    return lax.gather(x, jnp.asarray(idx).reshape(16, 1), _TAKE_DNUMS, (1,),
                      mode=lax.GatherScatterMode.PROMISE_IN_BOUNDS)


def _tree_sum(x, iota):
    for s in (1, 2, 4, 8):
        x = x + _lane_take(x, jnp.bitwise_xor(iota, s))
    return x


def _rsqrt_nr(x):
    i = lax.bitcast_convert_type(x, jnp.int32)
    i = jnp.int32(0x5F3759DF) - lax.shift_right_arithmetic(i, 1)
    y = lax.bitcast_convert_type(i, jnp.float32)
    half = x * 0.5
    for _ in range(3):
        y = y * (1.5 - half * y * y)
    return y


def _sc_body(uidx_hbm, iidx_hbm, uemb_hbm, iemb_hbm, out_hbm,
             uidx_s, urows_s,
             iidx0, iidx1, rows0, rows1, score0, score1,
             semu, semi0, semi1, semg0, semg1, semo0, semo1):
    wid = lax.axis_index("s") * 2 + lax.axis_index("c")
    b0 = wid * UPW

    iidx = (iidx0, iidx1)
    rows = (rows0, rows1)
    score = (score0, score1)
    semi = (semi0, semi1)
    semg = (semg0, semg1)
    semo = (semo0, semo1)

    pltpu.sync_copy(uidx_hbm.at[wid], uidx_s)
    for j in range(4):
        pltpu.async_copy(uemb_hbm.at[uidx_s.at[j]],
                         urows_s.at[pl.ds(j * 128, 128)], semu).wait()

    iota = lax.iota(jnp.int32, L)

    def start_idx(t, p):
        pltpu.async_copy(iidx_hbm.at[b0 + t], iidx[p], semi[p])

    def wait_idx(p):
        pltpu.make_async_copy(iidx_hbm.at[b0], iidx[p], semi[p]).wait()

    def start_gather(p):
        pltpu.async_copy(iemb_hbm.at[iidx[p].at[0]],
                         rows[p].at[pl.ds(0, 100)], semg[p])
        pltpu.async_copy(iemb_hbm.at[iidx[p].at[1]],
                         rows[p].at[pl.ds(100, 100)], semg[p])

    def wait_gather(p):
        pltpu.make_async_copy(iemb_hbm.at[pl.ds(0, K)], rows[p], semg[p]).wait()

    def start_out(t, p):
        pltpu.async_copy(score[p].at[pl.ds(0, K)], out_hbm.at[b0 + t], semo[p])

    def wait_out(p):
        pltpu.make_async_copy(score[p].at[pl.ds(0, K)], out_hbm.at[b0],
                              semo[p]).wait()

    def compute(t, p):
        rows_p, score_p = rows[p], score[p]
        u0 = urows_s[t, pl.ds(0, L)]
        u1 = urows_s[t, pl.ds(L, L)]
        n2u = _tree_sum(u0 * u0 + u1 * u1, iota)
        cu = _rsqrt_nr(jnp.maximum(n2u, 1e-24)) * (1.0 / TAU)
        for blk in range(KB):
            rr = jnp.minimum(iota + blk * L, K - 1)
            acc = jnp.zeros((L,), jnp.float32)
            n2 = jnp.zeros((L,), jnp.float32)
            for d in range(D):
                x = plsc.load_gather(rows_p, [rr, jnp.full((L,), d, jnp.int32)])
                ud = _lane_take(u0 if d < L else u1,
                                jnp.full((L,), d % L, jnp.int32))
                acc = acc + ud * x
                n2 = n2 + x * x
            r = _rsqrt_nr(jnp.maximum(n2, 1e-24))
            score_p[pl.ds(blk * L, L)] = acc * r * cu

    pltpu.sync_copy(iidx_hbm.at[b0], iidx[0])
    start_gather(0)
    start_idx(1, 1)

    def step(t, p, pn):
        wait_gather(p)

        @pl.when(t < UPW - 2)
        def _():
            start_idx(t + 2, p)

        @pl.when(t < UPW - 1)
        def _():
            wait_idx(pn)
            start_gather(pn)

        @pl.when(t >= 2)
        def _():
            wait_out(p)

        compute(t, p)
        start_out(t, p)

    def loop_body(g, carry):
        t = g * 2
        step(t, 0, 1)
        step(t + 1, 1, 0)
        return carry

    lax.fori_loop(0, UPW // 2, loop_body, 0)
    wait_out(0)
    wait_out(1)


@jax.jit
def _launch(user_idx, item_idx, uemb, iemb):
    uidx = user_idx.reshape(NW, 4, 128).astype(jnp.int32)
    iidx = item_idx.reshape(B, 2, 100).astype(jnp.int32)
    mesh = plsc.VectorSubcoreMesh(core_axis_name="c", subcore_axis_name="s")
    kern = functools.partial(
        pl.kernel,
        out_type=jax.ShapeDtypeStruct((B, K), jnp.float32),
        mesh=mesh,
        compiler_params=pltpu.CompilerParams(
            needs_layout_passes=False, use_tc_tiling_on_sc=False),
        scratch_types=[
            pltpu.VMEM((4, 128), jnp.int32),
            pltpu.VMEM((UPW, D), jnp.float32),
            pltpu.VMEM((2, 100), jnp.int32),
            pltpu.VMEM((2, 100), jnp.int32),
            pltpu.VMEM((K, D), jnp.float32),
            pltpu.VMEM((K, D), jnp.float32),
            pltpu.VMEM((KB * L,), jnp.float32),
            pltpu.VMEM((KB * L,), jnp.float32),
            pltpu.SemaphoreType.DMA,
            pltpu.SemaphoreType.DMA,
            pltpu.SemaphoreType.DMA,
            pltpu.SemaphoreType.DMA,
            pltpu.SemaphoreType.DMA,
            pltpu.SemaphoreType.DMA,
            pltpu.SemaphoreType.DMA,
        ],
    )(_sc_body)
    return kern(uidx, iidx, uemb, iemb)


def kernel(user_idx, item_idx, user_emb, item_emb):
    return _launch(user_idx, item_idx, user_emb, item_emb)
